# Initial kernel scaffold; baseline (speedup 1.0000x reference)
import jax
import jax.numpy as jnp
from jax.experimental import pallas as pl


def kernel(x_nchw, mean, std, w1, b1, w2, b2):
    raise NotImplementedError("write your pallas kernel here")


# speedup vs baseline: 1.6248x; 1.6248x over previous
import functools

import jax
import jax.numpy as jnp
from jax import lax
from jax.experimental import pallas as pl
from jax.experimental.pallas import tpu as pltpu

_LANE = 128

_CompilerParams = getattr(pltpu, "CompilerParams", None)
if _CompilerParams is None:
    _CompilerParams = getattr(pltpu, "TPUCompilerParams")


def _round_up(v, m):
    return ((v + m - 1) // m) * m


def _fused_kernel(x_ref, w1t_ref, b1_ref, w2_ref, b2_ref, probs_ref, acc_ref,
                  *, hw, chunk):
    acc_ref[...] = jnp.zeros_like(acc_ref)
    w1t = w1t_ref[...]
    b1 = b1_ref[...]

    def body(i, carry):
        start = pl.multiple_of(i * chunk, _LANE)
        xc = x_ref[0, :, pl.ds(start, chunk)]
        a = jnp.dot(w1t, xc, preferred_element_type=jnp.float32) + b1
        s = a * jnp.tanh(a) + a
        for j in range(chunk // _LANE):
            acc_ref[...] += s[:, j * _LANE:(j + 1) * _LANE]
        return carry

    lax.fori_loop(0, hw // chunk, body, 0)

    acc_t = acc_ref[...].T.astype(jnp.bfloat16)
    part = jnp.dot(acc_t, w2_ref[...], preferred_element_type=jnp.float32)
    logits = jnp.sum(part, axis=0, keepdims=True) + b2_ref[...]
    m = jnp.max(logits, axis=-1, keepdims=True)
    e = jnp.exp(logits - m)
    denom = jnp.sum(e, axis=-1, keepdims=True)
    probs_ref[0] = e * pl.reciprocal(denom, approx=True)


def kernel(x_nchw, mean, std, w1, b1, w2, b2):
    B, C, H, W = x_nchw.shape
    HW = H * W
    F = w1.shape[1]
    K = w2.shape[1]
    Fp = _round_up(F, _LANE)
    Kp = _round_up(K, _LANE)

    chunk = HW
    for cand in (1024, 512, 256, _LANE):
        if HW % cand == 0:
            chunk = cand
            break

    x = x_nchw.reshape(B, C, HW).astype(jnp.bfloat16)

    inv_std = 1.0 / std
    w1_fold = w1 * inv_std.reshape(C, 1)
    b1_fold = b1 - (mean * inv_std) @ w1
    w1t = (jnp.zeros((Fp, C), jnp.float32)
           .at[:F].set(0.5 * w1_fold.T).astype(jnp.bfloat16))
    b1c = jnp.zeros((Fp, 1), jnp.float32).at[:F].set(0.5 * b1_fold.reshape(F, 1))

    inv_hw = 1.0 / float(HW)
    w2p = (jnp.zeros((Fp, Kp), jnp.float32)
           .at[:F, :K].set(w2 * inv_hw).astype(jnp.bfloat16))
    b2p = jnp.full((1, Kp), -1e30, jnp.float32).at[:, :K].set(b2)

    kern = functools.partial(_fused_kernel, hw=HW, chunk=chunk)
    probs = pl.pallas_call(
        kern,
        out_shape=jax.ShapeDtypeStruct((B, 1, Kp), jnp.float32),
        grid_spec=pltpu.PrefetchScalarGridSpec(
            num_scalar_prefetch=0,
            grid=(B,),
            in_specs=[
                pl.BlockSpec((1, C, HW), lambda b: (b, 0, 0)),
                pl.BlockSpec((Fp, C), lambda b: (0, 0)),
                pl.BlockSpec((Fp, 1), lambda b: (0, 0)),
                pl.BlockSpec((Fp, Kp), lambda b: (0, 0)),
                pl.BlockSpec((1, Kp), lambda b: (0, 0)),
            ],
            out_specs=pl.BlockSpec((1, 1, Kp), lambda b: (b, 0, 0)),
            scratch_shapes=[pltpu.VMEM((Fp, _LANE), jnp.float32)],
        ),
        compiler_params=_CompilerParams(
            dimension_semantics=("parallel",),
            vmem_limit_bytes=48 * 1024 * 1024),
    )(x, w1t, b1c, w2p, b2p)

    return probs.reshape(B, Kp)[:, :K]


# speedup vs baseline: 1.8977x; 1.1680x over previous
import functools

import jax
import jax.numpy as jnp
from jax import lax
from jax.experimental import pallas as pl
from jax.experimental.pallas import tpu as pltpu

_LANE = 128

_CompilerParams = getattr(pltpu, "CompilerParams", None)
if _CompilerParams is None:
    _CompilerParams = getattr(pltpu, "TPUCompilerParams")


def _round_up(v, m):
    return ((v + m - 1) // m) * m


def _fused_kernel(x_ref, w1a_ref, w2_ref, lin_ref, probs_ref, acc_ref,
                  *, hw, chunk):
    acc_ref[...] = jnp.zeros_like(acc_ref)
    w1a = w1a_ref[...]
    n_slices = chunk // _LANE

    def body(i, carry):
        start = pl.multiple_of(i * chunk, _LANE)
        xc = x_ref[0, :, pl.ds(start, chunk)]
        a = jnp.dot(w1a, xc, preferred_element_type=jnp.float32)
        s = a * jnp.tanh(a)
        parts = [s[:, j * _LANE:(j + 1) * _LANE] for j in range(n_slices)]
        while len(parts) > 1:
            parts = ([parts[j] + parts[j + 1] for j in range(0, len(parts) - 1, 2)]
                     + ([parts[-1]] if len(parts) % 2 else []))
        acc_ref[...] += parts[0]
        return carry

    lax.fori_loop(0, hw // chunk, body, 0)

    acc_t = acc_ref[...].T.astype(jnp.bfloat16)
    part = jnp.dot(acc_t, w2_ref[...], preferred_element_type=jnp.float32)
    logits = jnp.sum(part, axis=0, keepdims=True) + lin_ref[0]
    m = jnp.max(logits, axis=-1, keepdims=True)
    e = jnp.exp(logits - m)
    denom = jnp.sum(e, axis=-1, keepdims=True)
    probs_ref[0] = e * pl.reciprocal(denom, approx=True)


def kernel(x_nchw, mean, std, w1, b1, w2, b2):
    B, C, H, W = x_nchw.shape
    HW = H * W
    Ca = C + 1
    F = w1.shape[1]
    K = w2.shape[1]
    Fp = _round_up(F, _LANE)
    Kp = _round_up(K, _LANE)

    chunk = HW
    for cand in (1024, 512, 256, _LANE):
        if HW % cand == 0:
            chunk = cand
            break

    x = x_nchw.reshape(B, C, HW).astype(jnp.bfloat16)
    x4 = jnp.concatenate([x, jnp.ones((B, 1, HW), jnp.bfloat16)], axis=1)

    inv_std = 1.0 / std
    w1_fold = w1 * inv_std.reshape(C, 1)
    b1_fold = b1 - (mean * inv_std) @ w1
    w1a = (jnp.zeros((Fp, Ca), jnp.float32)
           .at[:F, :C].set(0.5 * w1_fold.T)
           .at[:F, C].set(0.5 * b1_fold.reshape(F))
           .astype(jnp.bfloat16))

    inv_hw = 1.0 / float(HW)
    w2p = (jnp.zeros((Fp, Kp), jnp.float32)
           .at[:F, :K].set(w2 * inv_hw).astype(jnp.bfloat16))

    xs = jnp.sum(x4.astype(jnp.float32), axis=-1)
    w2s = w2 * inv_hw
    lin = (xs @ w1a.astype(jnp.float32)[:F].T) @ w2s + b2
    linp = (jnp.full((B, 1, Kp), -1e30, jnp.float32)
            .at[:, 0, :K].set(lin))

    kern = functools.partial(_fused_kernel, hw=HW, chunk=chunk)
    probs = pl.pallas_call(
        kern,
        out_shape=jax.ShapeDtypeStruct((B, 1, Kp), jnp.float32),
        grid_spec=pltpu.PrefetchScalarGridSpec(
            num_scalar_prefetch=0,
            grid=(B,),
            in_specs=[
                pl.BlockSpec((1, Ca, HW), lambda b: (b, 0, 0)),
                pl.BlockSpec((Fp, Ca), lambda b: (0, 0)),
                pl.BlockSpec((Fp, Kp), lambda b: (0, 0)),
                pl.BlockSpec((1, 1, Kp), lambda b: (b, 0, 0)),
            ],
            out_specs=pl.BlockSpec((1, 1, Kp), lambda b: (b, 0, 0)),
            scratch_shapes=[pltpu.VMEM((Fp, _LANE), jnp.float32)],
        ),
        compiler_params=_CompilerParams(
            dimension_semantics=("parallel",),
            vmem_limit_bytes=48 * 1024 * 1024),
    )(x4, w1a, w2p, linp)

    return probs.reshape(B, Kp)[:, :K]


# speedup vs baseline: 3.0220x; 1.5925x over previous
import functools

import jax
import jax.numpy as jnp
from jax import lax
from jax.experimental import pallas as pl
from jax.experimental.pallas import tpu as pltpu

_LANE = 128

_CompilerParams = getattr(pltpu, "CompilerParams", None)
if _CompilerParams is None:
    _CompilerParams = getattr(pltpu, "TPUCompilerParams")


def _round_up(v, m):
    return ((v + m - 1) // m) * m


def _fused_kernel(x_ref, w1a_ref, w2_ref, lin_ref, probs_ref, acc_ref,
                  *, hw, chunk, ipg):
    w1a = w1a_ref[...]
    n_slices = chunk // _LANE

    for g in range(ipg):
        acc_ref[...] = jnp.zeros_like(acc_ref)

        def body(i, carry):
            start = pl.multiple_of(i * chunk, _LANE)
            xc = x_ref[g, :, pl.ds(start, chunk)]
            a = jnp.dot(w1a, xc, preferred_element_type=jnp.float32)
            s = a * jnp.tanh(a)
            fold = s[:, 0:_LANE]
            for j in range(1, n_slices):
                fold = fold + s[:, j * _LANE:(j + 1) * _LANE]
            acc_ref[...] += fold
            return carry

        lax.fori_loop(0, hw // chunk, body, 0)

        acc_t = acc_ref[...].T.astype(jnp.bfloat16)
        part = jnp.dot(acc_t, w2_ref[...], preferred_element_type=jnp.float32)
        logits = jnp.sum(part, axis=0, keepdims=True) + lin_ref[g]
        m = jnp.max(logits, axis=-1, keepdims=True)
        e = jnp.exp(logits - m)
        denom = jnp.sum(e, axis=-1, keepdims=True)
        probs_ref[g] = e * pl.reciprocal(denom, approx=True)


def kernel(x_nchw, mean, std, w1, b1, w2, b2):
    B, C, H, W = x_nchw.shape
    HW = H * W
    Ca = C + 1
    F = w1.shape[1]
    K = w2.shape[1]
    Fp = _round_up(F, _LANE)
    Kp = _round_up(K, _LANE)
    ipg = 2 if B % 2 == 0 else 1

    chunk = HW
    for cand in (50176, 25088, 12544, 7168, 3584, 1024, 512, 256, _LANE):
        if HW % cand == 0:
            chunk = cand
            break

    x = x_nchw.reshape(B, C, HW).astype(jnp.bfloat16)
    x4 = jnp.concatenate([x, jnp.ones((B, 1, HW), jnp.bfloat16)], axis=1)

    inv_std = 1.0 / std
    w1_fold = w1 * inv_std.reshape(C, 1)
    b1_fold = b1 - (mean * inv_std) @ w1
    w1a = (jnp.zeros((Fp, Ca), jnp.float32)
           .at[:F, :C].set(0.5 * w1_fold.T)
           .at[:F, C].set(0.5 * b1_fold.reshape(F))
           .astype(jnp.bfloat16))

    inv_hw = 1.0 / float(HW)
    w2p = (jnp.zeros((Fp, Kp), jnp.float32)
           .at[:F, :K].set(w2 * inv_hw).astype(jnp.bfloat16))

    xs = jnp.sum(x4.astype(jnp.float32), axis=-1)
    w2s = w2 * inv_hw
    lin = (xs @ w1a.astype(jnp.float32)[:F].T) @ w2s + b2
    linp = (jnp.full((B, 1, Kp), -1e30, jnp.float32)
            .at[:, 0, :K].set(lin))

    kern = functools.partial(_fused_kernel, hw=HW, chunk=chunk, ipg=ipg)
    probs = pl.pallas_call(
        kern,
        out_shape=jax.ShapeDtypeStruct((B, 1, Kp), jnp.float32),
        grid_spec=pltpu.PrefetchScalarGridSpec(
            num_scalar_prefetch=0,
            grid=(B // ipg,),
            in_specs=[
                pl.BlockSpec((ipg, Ca, HW), lambda b: (b, 0, 0)),
                pl.BlockSpec((Fp, Ca), lambda b: (0, 0)),
                pl.BlockSpec((Fp, Kp), lambda b: (0, 0)),
                pl.BlockSpec((ipg, 1, Kp), lambda b: (b, 0, 0)),
            ],
            out_specs=pl.BlockSpec((ipg, 1, Kp), lambda b: (b, 0, 0)),
            scratch_shapes=[pltpu.VMEM((Fp, _LANE), jnp.float32)],
        ),
        compiler_params=_CompilerParams(
            dimension_semantics=("parallel",),
            vmem_limit_bytes=48 * 1024 * 1024),
    )(x4, w1a, w2p, linp)

    return probs.reshape(B, Kp)[:, :K]
